# baseline (speedup 1.0000x reference)
import functools

import jax
import jax.numpy as jnp
from jax import lax
from jax.experimental import pallas as pl
from jax.experimental.pallas import tpu as pltpu

_E = 16
_K = 8
_H = 2048
_I = 768
_T = 512

_IB = 256
_NI = _I // _IB


def _moe_tc_body(w_ref, x_ref, gate_ref, up_ref, down_ref, out_ref):
    e = pl.program_id(0)
    i = pl.program_id(1)

    @pl.when((e == 0) & (i == 0))
    def _init():
        out_ref[...] = jnp.zeros_like(out_ref)

    x = x_ref[...].astype(jnp.bfloat16)
    g = lax.dot_general(x, gate_ref[0].astype(jnp.bfloat16),
                        (((1,), (1,)), ((), ())),
                        preferred_element_type=jnp.float32)
    u = lax.dot_general(x, up_ref[0].astype(jnp.bfloat16),
                        (((1,), (1,)), ((), ())),
                        preferred_element_type=jnp.float32)
    f = (g * jax.nn.sigmoid(g)) * u
    w = w_ref[0, 0, :]
    fw = (f * w[:, None]).astype(jnp.bfloat16)
    out_ref[...] += lax.dot_general(
        fw, down_ref[0].astype(jnp.bfloat16), (((1,), (1,)), ((), ())),
        preferred_element_type=jnp.float32)


def _moe_tc(w_et, hidden, gate_proj, up_proj, down_proj):
    return pl.pallas_call(
        _moe_tc_body,
        grid=(_E, _NI),
        in_specs=[
            pl.BlockSpec((1, 1, _T), lambda e, i: (e, 0, 0)),
            pl.BlockSpec((_T, _H), lambda e, i: (0, 0)),
            pl.BlockSpec((1, _IB, _H), lambda e, i: (e, i, 0)),
            pl.BlockSpec((1, _IB, _H), lambda e, i: (e, i, 0)),
            pl.BlockSpec((1, _H, _IB), lambda e, i: (e, 0, i)),
        ],
        out_specs=pl.BlockSpec((_T, _H), lambda e, i: (0, 0)),
        out_shape=jax.ShapeDtypeStruct((_T, _H), jnp.float32),
    )(w_et, hidden, gate_proj, up_proj, down_proj)


def _routing_weights(router_logits):
    p = jax.nn.softmax(router_logits.astype(jnp.float32), axis=-1)
    top_w, top_idx = lax.top_k(p, _K)
    top_w = top_w / jnp.sum(top_w, axis=-1, keepdims=True)
    w = jnp.zeros((_T, _E), jnp.float32)
    w = w.at[jnp.arange(_T)[:, None], top_idx].set(top_w)
    return w


def kernel(hidden_states, router_logits, gate_proj, up_proj, down_proj):
    w_te = _routing_weights(router_logits)
    w_et = w_te.T.reshape(_E, 1, _T)
    return _moe_tc(w_et, hidden_states, gate_proj, up_proj, down_proj)


# speedup vs baseline: 1.1110x; 1.1110x over previous
import functools

import jax
import jax.numpy as jnp
from jax import lax
from jax.experimental import pallas as pl
from jax.experimental.pallas import tpu as pltpu

_E = 16
_K = 8
_H = 2048
_I = 768
_T = 512

_IB = 768
_NI = _I // _IB


def _moe_tc_body(w_ref, x_ref, gate_ref, up_ref, down_ref, out_ref):
    e = pl.program_id(0)
    i = pl.program_id(1)

    @pl.when((e == 0) & (i == 0))
    def _init():
        out_ref[...] = jnp.zeros_like(out_ref)

    x = x_ref[...].astype(jnp.bfloat16)
    g = lax.dot_general(x, gate_ref[0].astype(jnp.bfloat16),
                        (((1,), (1,)), ((), ())),
                        preferred_element_type=jnp.float32)
    u = lax.dot_general(x, up_ref[0].astype(jnp.bfloat16),
                        (((1,), (1,)), ((), ())),
                        preferred_element_type=jnp.float32)
    f = (g * jax.nn.sigmoid(g)) * u
    w = w_ref[0, 0, :]
    fw = (f * w[:, None]).astype(jnp.bfloat16)
    out_ref[...] += lax.dot_general(
        fw, down_ref[0].astype(jnp.bfloat16), (((1,), (1,)), ((), ())),
        preferred_element_type=jnp.float32)


def _moe_tc(w_et, hidden, gate_proj, up_proj, down_proj):
    return pl.pallas_call(
        _moe_tc_body,
        grid=(_E, _NI),
        in_specs=[
            pl.BlockSpec((1, 1, _T), lambda e, i: (e, 0, 0)),
            pl.BlockSpec((_T, _H), lambda e, i: (0, 0)),
            pl.BlockSpec((1, _IB, _H), lambda e, i: (e, i, 0)),
            pl.BlockSpec((1, _IB, _H), lambda e, i: (e, i, 0)),
            pl.BlockSpec((1, _H, _IB), lambda e, i: (e, 0, i)),
        ],
        out_specs=pl.BlockSpec((_T, _H), lambda e, i: (0, 0)),
        out_shape=jax.ShapeDtypeStruct((_T, _H), jnp.float32),
    )(w_et, hidden, gate_proj, up_proj, down_proj)


def _routing_weights(router_logits):
    p = jax.nn.softmax(router_logits.astype(jnp.float32), axis=-1)
    top_w, top_idx = lax.top_k(p, _K)
    top_w = top_w / jnp.sum(top_w, axis=-1, keepdims=True)
    w = jnp.zeros((_T, _E), jnp.float32)
    w = w.at[jnp.arange(_T)[:, None], top_idx].set(top_w)
    return w


def kernel(hidden_states, router_logits, gate_proj, up_proj, down_proj):
    w_te = _routing_weights(router_logits)
    w_et = w_te.T.reshape(_E, 1, _T)
    return _moe_tc(w_et, hidden_states, gate_proj, up_proj, down_proj)
